# Initial kernel scaffold; baseline (speedup 1.0000x reference)
import jax
import jax.numpy as jnp
from jax.experimental import pallas as pl


def kernel(h_neigh, h_self, edge_index, W_self, W_neigh):
    raise NotImplementedError("write your pallas kernel here")


# speedup vs baseline: 6.0052x; 6.0052x over previous
import functools

import jax
import jax.numpy as jnp
from jax import lax
from jax.experimental import pallas as pl
from jax.experimental.pallas import tpu as pltpu
from jax.experimental.pallas import tpu_sc as plsc

_NC = 2
_NS = 16
_NW = _NC * _NS
_K = 80
_WCHUNK = 624


def _sc_segment_sum(N, E, D, interpret=False):
    dh = D // 2
    ept = E // _NW
    nchunk = ept // _K
    rpt = N // _NS
    nzero = rpt // _K
    rzero = rpt - nzero * _K
    wchunk = (N // _NS) & ~7

    mesh = plsc.VectorSubcoreMesh(core_axis_name="c", subcore_axis_name="s")

    @functools.partial(
        pl.kernel,
        mesh=mesh,
        interpret=interpret,
        compiler_params=pltpu.CompilerParams(use_tc_tiling_on_sc=False),
        out_type=[
            jax.ShapeDtypeStruct((_NC, N, dh), jnp.float32),
            jax.ShapeDtypeStruct((_NC, N, dh), jnp.float32),
            jax.ShapeDtypeStruct((_NC, N), jnp.float32),
        ],
        scratch_types=[
            pltpu.VMEM((nchunk, _K), jnp.int32),
            pltpu.VMEM((nchunk, _K), jnp.int32),
            pltpu.VMEM((_K, dh), jnp.float32),
            pltpu.VMEM((_K, dh), jnp.float32),
            pltpu.VMEM((_K,), jnp.float32),
            pltpu.VMEM((N,), jnp.float32),
            pltpu.SemaphoreType.DMA,
            pltpu.VMEM_SHARED((N, dh), jnp.float32),
            pltpu.VMEM_SHARED((N,), jnp.float32),
        ],
    )
    def sc(hn0_hbm, hn1_hbm, src_hbm, dst_hbm, out0_hbm, out1_hbm, deg_hbm,
           src_v, dst_v, rows_v, zrows_v, ones_v, zdeg_v, sem, acc_sh, deg_sh):
        c = lax.axis_index("c")
        s = lax.axis_index("s")
        wid = c * _NS + s

        zero16 = jnp.zeros((16,), jnp.float32)

        def zrow(i, carry):
            for j in range(dh // 16):
                zrows_v[i, pl.ds(j * 16, 16)] = zero16
            return carry
        lax.fori_loop(0, _K, zrow, 0)

        for i in range(_K // 16):
            ones_v[pl.ds(i * 16, 16)] = jnp.ones((16,), jnp.float32)

        @pl.when(s == 0)
        def _():
            def zdeg(i, carry):
                zdeg_v[pl.ds(i * 16, 16)] = zero16
                return carry
            lax.fori_loop(0, N // 16, zdeg, 0)

        pltpu.sync_copy(src_hbm.at[wid], src_v)
        pltpu.sync_copy(dst_hbm.at[wid], dst_v)

        base_r = s * rpt
        wbase = s * wchunk
        tail = _NS * wchunk

        for half, (hn_hbm, out_hbm) in enumerate(
                [(hn0_hbm, out0_hbm), (hn1_hbm, out1_hbm)]):
            for q in range(nzero):
                pltpu.sync_copy(zrows_v, acc_sh.at[pl.ds(base_r + q * _K, _K)])
            if rzero:
                pltpu.sync_copy(zrows_v.at[pl.ds(0, rzero)],
                                acc_sh.at[pl.ds(base_r + nzero * _K, rzero)])
            if half == 0:
                @pl.when(s == 0)
                def _():
                    pltpu.sync_copy(zdeg_v, deg_sh)

            plsc.subcore_barrier()

            def body(j, carry):
                pltpu.async_copy(hn_hbm.at[src_v.at[j]], rows_v, sem).wait()
                pltpu.sync_copy(rows_v, acc_sh.at[dst_v.at[j]], add=True)
                if half == 0:
                    pltpu.sync_copy(ones_v, deg_sh.at[dst_v.at[j]], add=True)
                return carry
            lax.fori_loop(0, nchunk, body, 0)

            plsc.subcore_barrier()

            pltpu.sync_copy(acc_sh.at[pl.ds(wbase, wchunk)],
                            out_hbm.at[c, pl.ds(wbase, wchunk)])

            @pl.when(s == _NS - 1)
            def _():
                pltpu.sync_copy(acc_sh.at[pl.ds(tail, N - tail)],
                                out_hbm.at[c, pl.ds(tail, N - tail)])

            if half == 0:
                @pl.when(s == 0)
                def _():
                    pltpu.sync_copy(deg_sh, deg_hbm.at[c])
                plsc.subcore_barrier()

    return sc


def _tc_body(hs_ref, p0_ref, p1_ref, d0_ref, d1_ref, ws_ref, wn_ref, o_ref):
    dh = p0_ref.shape[2]
    d = d0_ref[...] + d1_ref[...]
    invd = 1.0 / jnp.maximum(d, 1.0)
    n0 = (p0_ref[0] + p0_ref[1]) * invd
    n1 = (p1_ref[0] + p1_ref[1]) * invd
    z = jnp.dot(hs_ref[...], ws_ref[...], preferred_element_type=jnp.float32)
    z = z + jnp.dot(n0, wn_ref[0:dh, :], preferred_element_type=jnp.float32)
    z = z + jnp.dot(n1, wn_ref[dh:, :], preferred_element_type=jnp.float32)
    z = jnp.maximum(z, 0.0)
    n2 = jnp.sum(z * z, axis=1, keepdims=True)
    inv = jnp.where(n2 > 0.0, lax.rsqrt(n2), 1.0)
    o_ref[...] = z * inv


def kernel(h_neigh, h_self, edge_index, W_self, W_neigh):
    N, D = h_neigh.shape
    E = edge_index.shape[1]
    dh = D // 2
    nchunk = E // (_NW * _K)

    ei = edge_index.astype(jnp.int32)
    src = ei[0].reshape(_NW, nchunk, _K)
    dst = ei[1].reshape(_NW, nchunk, _K)
    hn0 = h_neigh[:, :dh]
    hn1 = h_neigh[:, dh:]

    p0, p1, degs = _sc_segment_sum(N, E, D)(hn0, hn1, src, dst)
    d0 = degs[0].reshape(N, 1)
    d1 = degs[1].reshape(N, 1)

    blk = 1000
    grid = (N // blk,)
    out = pl.pallas_call(
        _tc_body,
        grid=grid,
        in_specs=[
            pl.BlockSpec((blk, D), lambda i: (i, 0)),
            pl.BlockSpec((_NC, blk, dh), lambda i: (0, i, 0)),
            pl.BlockSpec((_NC, blk, dh), lambda i: (0, i, 0)),
            pl.BlockSpec((blk, 1), lambda i: (i, 0)),
            pl.BlockSpec((blk, 1), lambda i: (i, 0)),
            pl.BlockSpec((D, D), lambda i: (0, 0)),
            pl.BlockSpec((D, D), lambda i: (0, 0)),
        ],
        out_specs=pl.BlockSpec((blk, D), lambda i: (i, 0)),
        out_shape=jax.ShapeDtypeStruct((N, D), jnp.float32),
    )(h_self, p0, p1, d0, d1, W_self, W_neigh)
    return out


# speedup vs baseline: 9.5738x; 1.5943x over previous
import functools

import jax
import jax.numpy as jnp
from jax import lax
from jax.experimental import pallas as pl
from jax.experimental.pallas import tpu as pltpu
from jax.experimental.pallas import tpu_sc as plsc

_NC = 2
_NS = 16
_NW = _NC * _NS
_K = 80
_WCHUNK = 624


def _sc_segment_sum(N, E, D, interpret=False):
    dh = D // 2
    ept = E // _NW
    nchunk = ept // _K
    rpt = N // _NS
    nzero = rpt // _K
    rzero = rpt - nzero * _K
    wchunk = (N // _NS) & ~7
    assert nchunk % 2 == 1 and nchunk >= 3

    mesh = plsc.VectorSubcoreMesh(core_axis_name="c", subcore_axis_name="s")

    @functools.partial(
        pl.kernel,
        mesh=mesh,
        interpret=interpret,
        compiler_params=pltpu.CompilerParams(use_tc_tiling_on_sc=False),
        out_type=[
            jax.ShapeDtypeStruct((_NC, N, dh), jnp.float32),
            jax.ShapeDtypeStruct((_NC, N, dh), jnp.float32),
            jax.ShapeDtypeStruct((_NC, N), jnp.float32),
        ],
        scratch_types=[
            pltpu.VMEM((nchunk, _K), jnp.int32),
            pltpu.VMEM((nchunk, _K), jnp.int32),
            pltpu.VMEM((_K, dh), jnp.float32),
            pltpu.VMEM((_K, dh), jnp.float32),
            pltpu.VMEM((_K, dh), jnp.float32),
            pltpu.VMEM((_K,), jnp.float32),
            pltpu.VMEM((N,), jnp.float32),
            pltpu.SemaphoreType.DMA,
            pltpu.SemaphoreType.DMA,
            pltpu.SemaphoreType.DMA,
            pltpu.VMEM_SHARED((N, dh), jnp.float32),
            pltpu.VMEM_SHARED((N,), jnp.float32),
        ],
    )
    def sc(hn0_hbm, hn1_hbm, src_hbm, dst_hbm, out0_hbm, out1_hbm, deg_hbm,
           src_v, dst_v, rowsa_v, rowsb_v, zrows_v, ones_v, zdeg_v,
           sema, semb, dsem, acc_sh, deg_sh):
        c = lax.axis_index("c")
        s = lax.axis_index("s")
        wid = c * _NS + s

        zero16 = jnp.zeros((16,), jnp.float32)

        pltpu.async_copy(src_hbm.at[wid], src_v, sema)
        pltpu.async_copy(dst_hbm.at[wid], dst_v, semb)

        def zrow(i, carry):
            for j in range(dh // 16):
                zrows_v[i, pl.ds(j * 16, 16)] = zero16
            return carry
        lax.fori_loop(0, _K, zrow, 0)

        for i in range(_K // 16):
            ones_v[pl.ds(i * 16, 16)] = jnp.ones((16,), jnp.float32)

        @pl.when(s == 0)
        def _():
            def zdeg(i, carry):
                zdeg_v[pl.ds(i * 16, 16)] = zero16
                return carry
            lax.fori_loop(0, N // 16, zdeg, 0)

        pltpu.make_async_copy(src_hbm.at[wid], src_v, sema).wait()
        pltpu.make_async_copy(dst_hbm.at[wid], dst_v, semb).wait()

        base_r = s * rpt
        wbase = s * wchunk
        tail = _NS * wchunk

        for half, (hn_hbm, out_hbm) in enumerate(
                [(hn0_hbm, out0_hbm), (hn1_hbm, out1_hbm)]):
            for q in range(nzero):
                pltpu.sync_copy(zrows_v, acc_sh.at[pl.ds(base_r + q * _K, _K)])
            if rzero:
                pltpu.sync_copy(zrows_v.at[pl.ds(0, rzero)],
                                acc_sh.at[pl.ds(base_r + nzero * _K, rzero)])
            if half == 0:
                @pl.when(s == 0)
                def _():
                    pltpu.sync_copy(zdeg_v, deg_sh)

            plsc.subcore_barrier()

            def gstart(j, buf, gsem):
                pltpu.async_copy(hn_hbm.at[src_v.at[j]], buf, gsem)

            def gwait(j, buf, gsem):
                pltpu.make_async_copy(hn_hbm.at[src_v.at[j]], buf, gsem).wait()

            def scat(j, buf):
                pltpu.sync_copy(buf, acc_sh.at[dst_v.at[j]], add=True)
                if half == 0:
                    pltpu.async_copy(ones_v, deg_sh.at[dst_v.at[j]], dsem,
                                     add=True)

            npairs = (nchunk - 1) // 2
            gstart(0, rowsa_v, sema)

            def body(t, carry):
                j0 = 2 * t
                gstart(j0 + 1, rowsb_v, semb)
                gwait(j0, rowsa_v, sema)
                scat(j0, rowsa_v)
                gstart(j0 + 2, rowsa_v, sema)
                gwait(j0 + 1, rowsb_v, semb)
                scat(j0 + 1, rowsb_v)
                return carry
            lax.fori_loop(0, npairs, body, 0)

            gwait(nchunk - 1, rowsa_v, sema)
            scat(nchunk - 1, rowsa_v)

            if half == 0:
                if ept == N:
                    pltpu.make_async_copy(deg_hbm.at[c], zdeg_v, dsem).wait()
                else:
                    def ddrain(j, carry):
                        pltpu.make_async_copy(
                            deg_hbm.at[c, pl.ds(0, _K)],
                            zdeg_v.at[pl.ds(0, _K)], dsem).wait()
                        return carry
                    lax.fori_loop(0, nchunk, ddrain, 0)

            plsc.subcore_barrier()

            pltpu.sync_copy(acc_sh.at[pl.ds(wbase, wchunk)],
                            out_hbm.at[c, pl.ds(wbase, wchunk)])

            @pl.when(s == _NS - 1)
            def _():
                pltpu.sync_copy(acc_sh.at[pl.ds(tail, N - tail)],
                                out_hbm.at[c, pl.ds(tail, N - tail)])

            if half == 0:
                @pl.when(s == 0)
                def _():
                    pltpu.sync_copy(deg_sh, deg_hbm.at[c])
                plsc.subcore_barrier()

    return sc


def _tc_body(hs_ref, p0_ref, p1_ref, d0_ref, d1_ref, ws_ref, wn_ref, o_ref):
    dh = p0_ref.shape[2]
    d = d0_ref[...] + d1_ref[...]
    invd = 1.0 / jnp.maximum(d, 1.0)
    n0 = (p0_ref[0] + p0_ref[1]) * invd
    n1 = (p1_ref[0] + p1_ref[1]) * invd
    z = jnp.dot(hs_ref[...], ws_ref[...], preferred_element_type=jnp.float32)
    z = z + jnp.dot(n0, wn_ref[0:dh, :], preferred_element_type=jnp.float32)
    z = z + jnp.dot(n1, wn_ref[dh:, :], preferred_element_type=jnp.float32)
    z = jnp.maximum(z, 0.0)
    n2 = jnp.sum(z * z, axis=1, keepdims=True)
    inv = jnp.where(n2 > 0.0, lax.rsqrt(n2), 1.0)
    o_ref[...] = z * inv


def kernel(h_neigh, h_self, edge_index, W_self, W_neigh):
    N, D = h_neigh.shape
    E = edge_index.shape[1]
    dh = D // 2
    nchunk = E // (_NW * _K)

    ei = edge_index.astype(jnp.int32)
    src = ei[0].reshape(_NW, nchunk, _K)
    dst = ei[1].reshape(_NW, nchunk, _K)
    hn0 = h_neigh[:, :dh]
    hn1 = h_neigh[:, dh:]

    p0, p1, degs = _sc_segment_sum(N, E, D)(hn0, hn1, src, dst)
    d0 = degs[0].reshape(N, 1)
    d1 = degs[1].reshape(N, 1)

    blk = 1000
    grid = (N // blk,)
    out = pl.pallas_call(
        _tc_body,
        grid=grid,
        in_specs=[
            pl.BlockSpec((blk, D), lambda i: (i, 0)),
            pl.BlockSpec((_NC, blk, dh), lambda i: (0, i, 0)),
            pl.BlockSpec((_NC, blk, dh), lambda i: (0, i, 0)),
            pl.BlockSpec((blk, 1), lambda i: (i, 0)),
            pl.BlockSpec((blk, 1), lambda i: (i, 0)),
            pl.BlockSpec((D, D), lambda i: (0, 0)),
            pl.BlockSpec((D, D), lambda i: (0, 0)),
        ],
        out_specs=pl.BlockSpec((blk, D), lambda i: (i, 0)),
        out_shape=jax.ShapeDtypeStruct((N, D), jnp.float32),
    )(h_self, p0, p1, d0, d1, W_self, W_neigh)
    return out


# speedup vs baseline: 11.2420x; 1.1742x over previous
import functools

import jax
import jax.numpy as jnp
from jax import lax
from jax.experimental import pallas as pl
from jax.experimental.pallas import tpu as pltpu
from jax.experimental.pallas import tpu_sc as plsc

_NC = 2
_NS = 16
_NW = _NC * _NS
_K = 80
_WCHUNK = 624


def _sc_segment_sum(N, E, D, interpret=False):
    dh = D // 2
    ept = E // _NW
    nchunk = ept // _K
    rpt = N // _NS
    nzero = rpt // _K
    rzero = rpt - nzero * _K
    wchunk = (N // _NS) & ~7
    assert nchunk >= 4

    mesh = plsc.VectorSubcoreMesh(core_axis_name="c", subcore_axis_name="s")

    @functools.partial(
        pl.kernel,
        mesh=mesh,
        interpret=interpret,
        compiler_params=pltpu.CompilerParams(use_tc_tiling_on_sc=False),
        out_type=[
            jax.ShapeDtypeStruct((_NC, N, dh), jnp.float32),
            jax.ShapeDtypeStruct((_NC, N, dh), jnp.float32),
            jax.ShapeDtypeStruct((_NC, N), jnp.float32),
        ],
        scratch_types=[
            pltpu.VMEM((nchunk, _K), jnp.int32),
            pltpu.VMEM((nchunk, _K), jnp.int32),
            [pltpu.VMEM((_K, dh), jnp.float32)] * 4,
            pltpu.VMEM((_K, dh), jnp.float32),
            pltpu.VMEM((_K,), jnp.float32),
            pltpu.VMEM((N,), jnp.float32),
            [pltpu.SemaphoreType.DMA] * 4,
            [pltpu.SemaphoreType.DMA] * 4,
            pltpu.SemaphoreType.DMA,
            pltpu.VMEM_SHARED((N, dh), jnp.float32),
            pltpu.VMEM_SHARED((N,), jnp.float32),
        ],
    )
    def sc(hn0_hbm, hn1_hbm, src_hbm, dst_hbm, out0_hbm, out1_hbm, deg_hbm,
           src_v, dst_v, bufs, zrows_v, ones_v, zdeg_v, gsems, ssems, dsem,
           acc_sh, deg_sh):
        c = lax.axis_index("c")
        s = lax.axis_index("s")
        wid = c * _NS + s

        zero16 = jnp.zeros((16,), jnp.float32)

        pltpu.async_copy(src_hbm.at[wid], src_v, gsems[0])
        pltpu.async_copy(dst_hbm.at[wid], dst_v, gsems[1])

        def zrow(i, carry):
            for j in range(dh // 16):
                zrows_v[i, pl.ds(j * 16, 16)] = zero16
            return carry
        lax.fori_loop(0, _K, zrow, 0)

        for i in range(_K // 16):
            ones_v[pl.ds(i * 16, 16)] = jnp.ones((16,), jnp.float32)

        @pl.when(s == 0)
        def _():
            def zdeg(i, carry):
                zdeg_v[pl.ds(i * 16, 16)] = zero16
                return carry
            lax.fori_loop(0, N // 16, zdeg, 0)

        pltpu.make_async_copy(src_hbm.at[wid], src_v, gsems[0]).wait()
        pltpu.make_async_copy(dst_hbm.at[wid], dst_v, gsems[1]).wait()

        base_r = s * rpt
        wbase = s * wchunk
        tail = _NS * wchunk

        for half, (hn_hbm, out_hbm) in enumerate(
                [(hn0_hbm, out0_hbm), (hn1_hbm, out1_hbm)]):
            for q in range(nzero):
                pltpu.sync_copy(zrows_v, acc_sh.at[pl.ds(base_r + q * _K, _K)])
            if rzero:
                pltpu.sync_copy(zrows_v.at[pl.ds(0, rzero)],
                                acc_sh.at[pl.ds(base_r + nzero * _K, rzero)])
            if half == 0:
                @pl.when(s == 0)
                def _():
                    pltpu.sync_copy(zdeg_v, deg_sh)

            plsc.subcore_barrier()

            def gstart(j, slot):
                pltpu.async_copy(hn_hbm.at[src_v.at[j]], bufs[slot],
                                 gsems[slot])

            def gwait(j, slot):
                pltpu.make_async_copy(hn_hbm.at[src_v.at[j]], bufs[slot],
                                      gsems[slot]).wait()

            def sstart(j, slot):
                pltpu.async_copy(bufs[slot], acc_sh.at[dst_v.at[j]],
                                 ssems[slot], add=True)
                if half == 0:
                    pltpu.async_copy(ones_v, deg_sh.at[dst_v.at[j]], dsem,
                                     add=True)

            def swait(j, slot):
                pltpu.make_async_copy(bufs[slot], acc_sh.at[dst_v.at[j]],
                                      ssems[slot]).wait()

            ngroup = nchunk // 4
            nrest = nchunk - ngroup * 4

            for slot in range(4):
                gstart(slot, slot)

            def body(t, carry):
                j0 = 4 * t
                for slot in range(4):
                    gwait(j0 + slot, slot)
                    sstart(j0 + slot, slot)
                for slot in range(4):
                    swait(j0 + slot, slot)
                    nj = j0 + 4 + slot

                    @pl.when(nj < nchunk)
                    def _():
                        gstart(nj, slot)
                return carry
            lax.fori_loop(0, ngroup, body, 0)

            for r in range(nrest):
                j = ngroup * 4 + r
                gwait(j, r)
                sstart(j, r)
            for r in range(nrest):
                swait(ngroup * 4 + r, r)

            if half == 0:
                if ept == N:
                    pltpu.make_async_copy(deg_hbm.at[c], zdeg_v, dsem).wait()
                else:
                    def ddrain(j, carry):
                        pltpu.make_async_copy(
                            deg_hbm.at[c, pl.ds(0, _K)],
                            zdeg_v.at[pl.ds(0, _K)], dsem).wait()
                        return carry
                    lax.fori_loop(0, nchunk, ddrain, 0)

            plsc.subcore_barrier()

            pltpu.sync_copy(acc_sh.at[pl.ds(wbase, wchunk)],
                            out_hbm.at[c, pl.ds(wbase, wchunk)])

            @pl.when(s == _NS - 1)
            def _():
                pltpu.sync_copy(acc_sh.at[pl.ds(tail, N - tail)],
                                out_hbm.at[c, pl.ds(tail, N - tail)])

            if half == 0:
                @pl.when(s == 0)
                def _():
                    pltpu.sync_copy(deg_sh, deg_hbm.at[c])
                plsc.subcore_barrier()

    return sc


def _tc_body(hs_ref, p0_ref, p1_ref, d0_ref, d1_ref, ws_ref, wn_ref, o_ref):
    dh = p0_ref.shape[2]
    d = d0_ref[...] + d1_ref[...]
    invd = 1.0 / jnp.maximum(d, 1.0)
    n0 = (p0_ref[0] + p0_ref[1]) * invd
    n1 = (p1_ref[0] + p1_ref[1]) * invd
    z = jnp.dot(hs_ref[...], ws_ref[...], preferred_element_type=jnp.float32)
    z = z + jnp.dot(n0, wn_ref[0:dh, :], preferred_element_type=jnp.float32)
    z = z + jnp.dot(n1, wn_ref[dh:, :], preferred_element_type=jnp.float32)
    z = jnp.maximum(z, 0.0)
    n2 = jnp.sum(z * z, axis=1, keepdims=True)
    inv = jnp.where(n2 > 0.0, lax.rsqrt(n2), 1.0)
    o_ref[...] = z * inv


def kernel(h_neigh, h_self, edge_index, W_self, W_neigh):
    N, D = h_neigh.shape
    E = edge_index.shape[1]
    dh = D // 2
    nchunk = E // (_NW * _K)

    ei = edge_index.astype(jnp.int32)
    src = ei[0].reshape(_NW, nchunk, _K)
    dst = ei[1].reshape(_NW, nchunk, _K)
    hn0 = h_neigh[:, :dh]
    hn1 = h_neigh[:, dh:]

    p0, p1, degs = _sc_segment_sum(N, E, D)(hn0, hn1, src, dst)
    d0 = degs[0].reshape(N, 1)
    d1 = degs[1].reshape(N, 1)

    blk = 1000
    grid = (N // blk,)
    out = pl.pallas_call(
        _tc_body,
        grid=grid,
        in_specs=[
            pl.BlockSpec((blk, D), lambda i: (i, 0)),
            pl.BlockSpec((_NC, blk, dh), lambda i: (0, i, 0)),
            pl.BlockSpec((_NC, blk, dh), lambda i: (0, i, 0)),
            pl.BlockSpec((blk, 1), lambda i: (i, 0)),
            pl.BlockSpec((blk, 1), lambda i: (i, 0)),
            pl.BlockSpec((D, D), lambda i: (0, 0)),
            pl.BlockSpec((D, D), lambda i: (0, 0)),
        ],
        out_specs=pl.BlockSpec((blk, D), lambda i: (i, 0)),
        out_shape=jax.ShapeDtypeStruct((N, D), jnp.float32),
    )(h_self, p0, p1, d0, d1, W_self, W_neigh)
    return out


# speedup vs baseline: 12.9368x; 1.1508x over previous
import functools

import jax
import jax.numpy as jnp
from jax import lax
from jax.experimental import pallas as pl
from jax.experimental.pallas import tpu as pltpu
from jax.experimental.pallas import tpu_sc as plsc

_NC = 2
_NS = 16
_NW = _NC * _NS
_K = 80
_NBUF = 4


def _sc_segment_sum(N, E, D):
    dh = D // 2
    ept = E // _NW
    nchunk = ept // _K
    rpt = N // _NS
    nzero = rpt // _K
    rzero = rpt - nzero * _K
    wchunk = (N // _NS) & ~7
    assert nchunk >= _NBUF

    mesh = plsc.VectorSubcoreMesh(core_axis_name="c", subcore_axis_name="s")

    @functools.partial(
        pl.kernel,
        mesh=mesh,
        compiler_params=pltpu.CompilerParams(use_tc_tiling_on_sc=False),
        out_type=[
            jax.ShapeDtypeStruct((_NC, N, D), jnp.float32),
            jax.ShapeDtypeStruct((_NC, N), jnp.float32),
        ],
        scratch_types=[
            pltpu.VMEM((nchunk, _K), jnp.int32),
            pltpu.VMEM((nchunk, _K), jnp.int32),
            pltpu.VMEM((nchunk, _K), jnp.int32),
            [pltpu.VMEM((_K, dh), jnp.float32)] * _NBUF,
            pltpu.VMEM((_K, dh), jnp.float32),
            pltpu.VMEM((_K,), jnp.float32),
            pltpu.VMEM((N,), jnp.float32),
            [pltpu.SemaphoreType.DMA] * _NBUF,
            [pltpu.SemaphoreType.DMA] * _NBUF,
            pltpu.SemaphoreType.DMA,
            pltpu.VMEM_SHARED((N, dh), jnp.float32),
            pltpu.VMEM_SHARED((N,), jnp.float32),
        ],
    )
    def sc(hn_hbm, src0_hbm, src1_hbm, dst_hbm, out_hbm, deg_hbm,
           src0_v, src1_v, dst_v, bufs, zrows_v, ones_v, zdeg_v,
           gsems, ssems, dsem, acc_sh, deg_sh):
        c = lax.axis_index("c")
        s = lax.axis_index("s")
        wid = c * _NS + s

        zero16 = jnp.zeros((16,), jnp.float32)

        pltpu.async_copy(src0_hbm.at[wid], src0_v, gsems[0])
        pltpu.async_copy(src1_hbm.at[wid], src1_v, gsems[1])
        pltpu.async_copy(dst_hbm.at[wid], dst_v, gsems[2])

        def zrow(i, carry):
            for j in range(dh // 16):
                zrows_v[i, pl.ds(j * 16, 16)] = zero16
            return carry
        lax.fori_loop(0, _K, zrow, 0)

        for i in range(_K // 16):
            ones_v[pl.ds(i * 16, 16)] = jnp.ones((16,), jnp.float32)

        @pl.when(s == 0)
        def _():
            def zdeg(i, carry):
                zdeg_v[pl.ds(i * 16, 16)] = zero16
                return carry
            lax.fori_loop(0, N // 16, zdeg, 0)

        pltpu.make_async_copy(src0_hbm.at[wid], src0_v, gsems[0]).wait()
        pltpu.make_async_copy(src1_hbm.at[wid], src1_v, gsems[1]).wait()
        pltpu.make_async_copy(dst_hbm.at[wid], dst_v, gsems[2]).wait()

        base_r = s * rpt
        wbase = s * wchunk
        tail = _NS * wchunk

        for half, src_v in enumerate([src0_v, src1_v]):
            for q in range(nzero):
                pltpu.sync_copy(zrows_v, acc_sh.at[pl.ds(base_r + q * _K, _K)])
            if rzero:
                pltpu.sync_copy(zrows_v.at[pl.ds(0, rzero)],
                                acc_sh.at[pl.ds(base_r + nzero * _K, rzero)])
            if half == 0:
                @pl.when(s == 0)
                def _():
                    pltpu.sync_copy(zdeg_v, deg_sh)

            plsc.subcore_barrier()

            def gstart(j, slot):
                pltpu.async_copy(hn_hbm.at[src_v.at[j]], bufs[slot],
                                 gsems[slot])

            def gwait(j, slot):
                pltpu.make_async_copy(hn_hbm.at[src_v.at[j]], bufs[slot],
                                      gsems[slot]).wait()

            def sstart(j, slot):
                pltpu.async_copy(bufs[slot], acc_sh.at[dst_v.at[j]],
                                 ssems[slot], add=True)
                if half == 0:
                    pltpu.async_copy(ones_v, deg_sh.at[dst_v.at[j]], dsem,
                                     add=True)

            def swait(j, slot):
                pltpu.make_async_copy(bufs[slot], acc_sh.at[dst_v.at[j]],
                                      ssems[slot]).wait()

            ngroup = nchunk // _NBUF
            nrest = nchunk - ngroup * _NBUF

            for slot in range(_NBUF):
                gstart(slot, slot)

            def body(t, carry):
                j0 = _NBUF * t
                for slot in range(_NBUF):
                    gwait(j0 + slot, slot)
                    sstart(j0 + slot, slot)
                for slot in range(_NBUF):
                    swait(j0 + slot, slot)
                    nj = j0 + _NBUF + slot

                    @pl.when(nj < nchunk)
                    def _():
                        gstart(nj, slot)
                return carry
            lax.fori_loop(0, ngroup, body, 0)

            for r in range(nrest):
                j = ngroup * _NBUF + r
                gwait(j, r)
                sstart(j, r)
            for r in range(nrest):
                swait(ngroup * _NBUF + r, r)

            if half == 0:
                if ept == N:
                    pltpu.make_async_copy(deg_hbm.at[c], zdeg_v, dsem).wait()
                else:
                    def ddrain(j, carry):
                        pltpu.make_async_copy(
                            deg_hbm.at[c, pl.ds(0, _K)],
                            zdeg_v.at[pl.ds(0, _K)], dsem).wait()
                        return carry
                    lax.fori_loop(0, nchunk, ddrain, 0)

            plsc.subcore_barrier()

            cols = pl.ds(half * dh, dh)
            pltpu.sync_copy(acc_sh.at[pl.ds(wbase, wchunk)],
                            out_hbm.at[c, pl.ds(wbase, wchunk), cols])

            @pl.when(s == _NS - 1)
            def _():
                pltpu.sync_copy(acc_sh.at[pl.ds(tail, N - tail)],
                                out_hbm.at[c, pl.ds(tail, N - tail), cols])

            if half == 0:
                @pl.when(s == 0)
                def _():
                    pltpu.sync_copy(deg_sh, deg_hbm.at[c])
                plsc.subcore_barrier()

    return sc


def _tc_body(hs_ref, p_ref, d0_ref, d1_ref, ws_ref, wn_ref, o_ref):
    d = d0_ref[...] + d1_ref[...]
    invd = 1.0 / jnp.maximum(d, 1.0)
    neigh = (p_ref[0] + p_ref[1]) * invd
    z = jnp.dot(hs_ref[...], ws_ref[...], preferred_element_type=jnp.float32)
    z = z + jnp.dot(neigh, wn_ref[...], preferred_element_type=jnp.float32)
    z = jnp.maximum(z, 0.0)
    n2 = jnp.sum(z * z, axis=1, keepdims=True)
    inv = jnp.where(n2 > 0.0, lax.rsqrt(n2), 1.0)
    o_ref[...] = z * inv


def kernel(h_neigh, h_self, edge_index, W_self, W_neigh):
    N, D = h_neigh.shape
    dh = D // 2
    E = edge_index.shape[1]
    nchunk = E // (_NW * _K)

    ei = edge_index.astype(jnp.int32)
    src2 = ei[0] * 2
    src0 = src2.reshape(_NW, nchunk, _K)
    src1 = (src2 + 1).reshape(_NW, nchunk, _K)
    dst = ei[1].reshape(_NW, nchunk, _K)
    hn2 = h_neigh.reshape(2 * N, dh)

    parts, degs = _sc_segment_sum(N, E, D)(hn2, src0, src1, dst)
    d0 = degs[0].reshape(N, 1)
    d1 = degs[1].reshape(N, 1)

    blk = 2000
    grid = (N // blk,)
    out = pl.pallas_call(
        _tc_body,
        grid=grid,
        in_specs=[
            pl.BlockSpec((blk, D), lambda i: (i, 0)),
            pl.BlockSpec((_NC, blk, D), lambda i: (0, i, 0)),
            pl.BlockSpec((blk, 1), lambda i: (i, 0)),
            pl.BlockSpec((blk, 1), lambda i: (i, 0)),
            pl.BlockSpec((D, D), lambda i: (0, 0)),
            pl.BlockSpec((D, D), lambda i: (0, 0)),
        ],
        out_specs=pl.BlockSpec((blk, D), lambda i: (i, 0)),
        out_shape=jax.ShapeDtypeStruct((N, D), jnp.float32),
    )(h_self, parts, d0, d1, W_self, W_neigh)
    return out


# speedup vs baseline: 13.3993x; 1.0358x over previous
import functools

import jax
import jax.numpy as jnp
from jax import lax
from jax.experimental import pallas as pl
from jax.experimental.pallas import tpu as pltpu
from jax.experimental.pallas import tpu_sc as plsc

_NC = 2
_NS = 16
_NW = _NC * _NS
_K = 80
_NBUF = 3


def _sc_segment_sum(N, E, D):
    ept = E // _NW
    nchunk = ept // _K
    rpt = N // _NS
    nzero = rpt // _K
    rzero = rpt - nzero * _K
    wchunk = (N // _NS) & ~7
    dtail = N - _NS * wchunk
    assert nchunk >= _NBUF

    mesh = plsc.VectorSubcoreMesh(core_axis_name="c", subcore_axis_name="s")

    @functools.partial(
        pl.kernel,
        mesh=mesh,
        compiler_params=pltpu.CompilerParams(use_tc_tiling_on_sc=False),
        out_type=[
            jax.ShapeDtypeStruct((_NC, N, D), jnp.float32),
            jax.ShapeDtypeStruct((_NC, N), jnp.float32),
        ],
        scratch_types=[
            pltpu.VMEM((nchunk, _K), jnp.int32),
            [pltpu.VMEM((_K,), jnp.int32)] * _NBUF,
            [pltpu.VMEM((_K,), jnp.int32)] * _NBUF,
            [pltpu.VMEM((_K, D), jnp.float32)] * _NBUF,
            pltpu.VMEM((_K,), jnp.float32),
            pltpu.VMEM((wchunk + 16,), jnp.float32),
            [pltpu.SemaphoreType.DMA] * _NBUF,
            [pltpu.SemaphoreType.DMA] * _NBUF,
            pltpu.SemaphoreType.DMA,
            pltpu.VMEM_SHARED((N, D), jnp.float32),
            pltpu.VMEM_SHARED((N,), jnp.float32),
        ],
    )
    def sc(hn_hbm, sd_hbm, out_hbm, deg_hbm,
           sd_v, src_r, dst_r, bufs, ones_v, zdeg_v,
           gsems, ssems, zsem, acc_sh, deg_sh):
        c = lax.axis_index("c")
        s = lax.axis_index("s")
        wid = c * _NS + s

        zero16 = jnp.zeros((16,), jnp.float32)
        mask16 = jnp.full((16,), 0xFFFF, jnp.int32)
        sh16 = jnp.full((16,), 16, jnp.int32)

        pltpu.async_copy(sd_hbm.at[wid], sd_v, gsems[0])

        def zrow(i, carry):
            for j in range(D // 16):
                bufs[0][i, pl.ds(j * 16, 16)] = zero16
            return carry
        lax.fori_loop(0, _K, zrow, 0)

        for i in range(_K // 16):
            ones_v[pl.ds(i * 16, 16)] = jnp.ones((16,), jnp.float32)

        def zdeg(i, carry):
            zdeg_v[pl.ds(i * 16, 16)] = zero16
            return carry
        lax.fori_loop(0, (wchunk + 16) // 16, zdeg, 0)

        base_r = s * rpt
        for q in range(nzero):
            pltpu.async_copy(bufs[0], acc_sh.at[pl.ds(base_r + q * _K, _K)],
                             zsem)
        if rzero:
            pltpu.async_copy(bufs[0].at[pl.ds(0, rzero)],
                             acc_sh.at[pl.ds(base_r + nzero * _K, rzero)],
                             zsem)

        wbase = s * wchunk
        pltpu.sync_copy(zdeg_v.at[pl.ds(0, wchunk)],
                        deg_sh.at[pl.ds(wbase, wchunk)])

        @pl.when(s == _NS - 1)
        def _():
            pltpu.sync_copy(zdeg_v.at[pl.ds(0, dtail)],
                            deg_sh.at[pl.ds(_NS * wchunk, dtail)])

        pltpu.make_async_copy(sd_hbm.at[wid], sd_v, gsems[0]).wait()

        def unpack(j, slot):
            for q in range(_K // 16):
                v = sd_v[j, pl.ds(q * 16, 16)]
                src_r[slot][pl.ds(q * 16, 16)] = lax.bitwise_and(v, mask16)
                dst_r[slot][pl.ds(q * 16, 16)] = \
                    lax.shift_right_logical(v, sh16)

        for q in range(nzero):
            pltpu.make_async_copy(bufs[0],
                                  acc_sh.at[pl.ds(base_r + q * _K, _K)],
                                  zsem).wait()
        if rzero:
            pltpu.make_async_copy(bufs[0].at[pl.ds(0, rzero)],
                                  acc_sh.at[pl.ds(base_r + nzero * _K,
                                                  rzero)],
                                  zsem).wait()

        def gstart(slot):
            pltpu.async_copy(hn_hbm.at[src_r[slot]], bufs[slot], gsems[slot])

        def gwait(slot):
            pltpu.make_async_copy(hn_hbm.at[src_r[slot]], bufs[slot],
                                  gsems[slot]).wait()

        def sstart(slot):
            pltpu.async_copy(bufs[slot], acc_sh.at[dst_r[slot]],
                             ssems[slot], add=True)
            pltpu.async_copy(ones_v, deg_sh.at[dst_r[slot]], ssems[slot],
                             add=True)

        def swait(slot):
            pltpu.make_async_copy(bufs[slot], acc_sh.at[dst_r[slot]],
                                  ssems[slot]).wait()
            pltpu.make_async_copy(ones_v, deg_sh.at[dst_r[slot]],
                                  ssems[slot]).wait()

        for slot in range(_NBUF):
            unpack(slot, slot)
            gstart(slot)

        plsc.subcore_barrier()

        ngroup = nchunk // _NBUF
        nrest = nchunk - ngroup * _NBUF

        def body(t, carry):
            j0 = _NBUF * t
            for slot in range(_NBUF):
                gwait(slot)
                sstart(slot)
            for slot in range(_NBUF):
                swait(slot)
                nj = j0 + _NBUF + slot

                @pl.when(nj < nchunk)
                def _():
                    unpack(nj, slot)
                    gstart(slot)
            return carry
        lax.fori_loop(0, ngroup, body, 0)

        for r in range(nrest):
            gwait(r)
            sstart(r)
        for r in range(nrest):
            swait(r)

        plsc.subcore_barrier()

        tail = _NS * wchunk
        pltpu.sync_copy(acc_sh.at[pl.ds(wbase, wchunk)],
                        out_hbm.at[c, pl.ds(wbase, wchunk)])

        @pl.when(s == _NS - 1)
        def _():
            pltpu.sync_copy(acc_sh.at[pl.ds(tail, N - tail)],
                            out_hbm.at[c, pl.ds(tail, N - tail)])

        @pl.when(s == 0)
        def _():
            pltpu.sync_copy(deg_sh, deg_hbm.at[c])

    return sc


def _tc_body(hs_ref, p_ref, d0_ref, d1_ref, ws_ref, wn_ref, o_ref):
    d = d0_ref[...] + d1_ref[...]
    invd = 1.0 / jnp.maximum(d, 1.0)
    neigh = (p_ref[0] + p_ref[1]) * invd
    z = jnp.dot(hs_ref[...], ws_ref[...], preferred_element_type=jnp.float32)
    z = z + jnp.dot(neigh, wn_ref[...], preferred_element_type=jnp.float32)
    z = jnp.maximum(z, 0.0)
    n2 = jnp.sum(z * z, axis=1, keepdims=True)
    inv = jnp.where(n2 > 0.0, lax.rsqrt(n2), 1.0)
    o_ref[...] = z * inv


def kernel(h_neigh, h_self, edge_index, W_self, W_neigh):
    N, D = h_neigh.shape
    E = edge_index.shape[1]
    nchunk = E // (_NW * _K)

    ei = edge_index.astype(jnp.int32)
    packed = (ei[0] | (ei[1] << 16)).reshape(_NW, nchunk, _K)

    parts, degs = _sc_segment_sum(N, E, D)(h_neigh, packed)
    d0 = degs[0].reshape(N, 1)
    d1 = degs[1].reshape(N, 1)

    blk = 2000
    grid = (N // blk,)
    out = pl.pallas_call(
        _tc_body,
        grid=grid,
        in_specs=[
            pl.BlockSpec((blk, D), lambda i: (i, 0)),
            pl.BlockSpec((_NC, blk, D), lambda i: (0, i, 0)),
            pl.BlockSpec((blk, 1), lambda i: (i, 0)),
            pl.BlockSpec((blk, 1), lambda i: (i, 0)),
            pl.BlockSpec((D, D), lambda i: (0, 0)),
            pl.BlockSpec((D, D), lambda i: (0, 0)),
        ],
        out_specs=pl.BlockSpec((blk, D), lambda i: (i, 0)),
        out_shape=jax.ShapeDtypeStruct((N, D), jnp.float32),
    )(h_self, parts, d0, d1, W_self, W_neigh)
    return out
